# baseline (speedup 1.0000x reference)
import functools

import jax
import jax.numpy as jnp
from jax import lax
from jax.experimental import pallas as pl
from jax.experimental.pallas import tpu as pltpu
from jax.experimental.pallas import tpu_sc as plsc

N = 10000
E = 320000
D = 128
DE = 16
B = 64

NC = 2
NS = 16
NW = NC * NS
EPW = E // NW
CH = 80
NCHUNK = EPW // CH
ZCH = 80
NZC = N // ZCH
ZPT = (NZC + NS - 1) // NS

@functools.cache
def _mesh():
    return plsc.VectorSubcoreMesh(core_axis_name="c", subcore_axis_name="s",
                                  num_cores=NC, num_subcores=NS)


def _gather_body(tdst, tsrc, dst3, src3, s_out,
                 idxd, idxs, ba0, bb0, ba1, bb1, ob0, ob1,
                 sa0, sb0, sa1, sb1, sw0, sw1):
    wid = lax.axis_index("s") * NC + lax.axis_index("c")
    base = wid * EPW

    pltpu.sync_copy(dst3.at[wid], idxd)
    pltpu.sync_copy(src3.at[wid], idxs)

    sets = ((ba0, bb0, ob0, sa0, sb0, sw0), (ba1, bb1, ob1, sa1, sb1, sw1))

    def issue(j, ba, bb, sa, sb):
        pltpu.async_copy(tdst.at[idxd.at[j]], ba, sa)
        pltpu.async_copy(tsrc.at[idxs.at[j]], bb, sb)

    def step(j, ba, bb, ob, sa, sb, sw):
        pltpu.make_async_copy(tdst.at[idxd.at[j]], ba, sa).wait()
        pltpu.make_async_copy(tsrc.at[idxs.at[j]], bb, sb).wait()

        @pl.when(j >= 2)
        def _():
            pltpu.make_async_copy(
                ob, s_out.at[pl.ds(base + (j - 2) * CH, CH)], sw).wait()

        mhi = jnp.int32(-65536)
        half = jnp.int32(32768)

        def row(r, c2):
            for c in range(D // 16):
                sl = pl.ds(c * 16, 16)
                va = ba[r, sl]
                vb = bb[r, sl]
                alo = lax.bitcast_convert_type(lax.shift_left(va, 16),
                                               jnp.float32)
                ahi = lax.bitcast_convert_type(jnp.bitwise_and(va, mhi),
                                               jnp.float32)
                blo = lax.bitcast_convert_type(lax.shift_left(vb, 16),
                                               jnp.float32)
                bhi = lax.bitcast_convert_type(jnp.bitwise_and(vb, mhi),
                                               jnp.float32)
                slo = lax.bitcast_convert_type(alo + blo, jnp.int32)
                shi = lax.bitcast_convert_type(ahi + bhi, jnp.int32)
                lo16 = lax.shift_right_logical(slo + half, 16)
                hi16 = jnp.bitwise_and(shi + half, mhi)
                ob[r, sl] = jnp.bitwise_or(hi16, lo16)
            return c2

        lax.fori_loop(0, CH, row, 0, unroll=2)

        @pl.when(j + 2 < NCHUNK)
        def _():
            issue(j + 2, ba, bb, sa, sb)

        pltpu.async_copy(ob, s_out.at[pl.ds(base + j * CH, CH)], sw)

    issue(0, *sets[0][:2], *sets[0][3:5])
    issue(1, *sets[1][:2], *sets[1][3:5])

    def pair(j2, carry):
        step(2 * j2, *sets[0])
        step(2 * j2 + 1, *sets[1])
        return carry

    lax.fori_loop(0, NCHUNK // 2, pair, 0)
    step(NCHUNK - 1, *sets[0])

    pltpu.make_async_copy(
        ob1, s_out.at[pl.ds(base + (NCHUNK - 2) * CH, CH)], sw1).wait()
    pltpu.make_async_copy(
        ob0, s_out.at[pl.ds(base + (NCHUNK - 1) * CH, CH)], sw0).wait()


@functools.cache
def _gather_kernel():
    return pl.kernel(
        _gather_body,
        out_type=jax.ShapeDtypeStruct((E, D), jnp.int32),
        mesh=_mesh(),
        scratch_types=[
            pltpu.VMEM((NCHUNK, CH), jnp.int32),
            pltpu.VMEM((NCHUNK, CH), jnp.int32),
            pltpu.VMEM((CH, D), jnp.int32),
            pltpu.VMEM((CH, D), jnp.int32),
            pltpu.VMEM((CH, D), jnp.int32),
            pltpu.VMEM((CH, D), jnp.int32),
            pltpu.VMEM((CH, D), jnp.int32),
            pltpu.VMEM((CH, D), jnp.int32),
            pltpu.SemaphoreType.DMA,
            pltpu.SemaphoreType.DMA,
            pltpu.SemaphoreType.DMA,
            pltpu.SemaphoreType.DMA,
            pltpu.SemaphoreType.DMA,
            pltpu.SemaphoreType.DMA,
        ],
    )


def _gather(td, ts, dst3, src3):
    return _gather_kernel()(td, ts, dst3, src3)


def _scatter_body(msg, dst3, out, idxd, mb0, mb1, agg, ls0, ls1):
    cid = lax.axis_index("c")
    sid = lax.axis_index("s")
    wid = sid * NC + cid
    base = wid * EPW

    pltpu.sync_copy(dst3.at[wid], idxd)

    def zrow(r, c2):
        for c in range(D // 16):
            mb0[r, pl.ds(c * 16, 16)] = jnp.zeros((16,), jnp.float32)
        return c2

    lax.fori_loop(0, ZCH, zrow, 0)
    for k in range(ZPT):
        zc = sid * ZPT + k

        @pl.when(zc < NZC)
        def _():
            pltpu.sync_copy(mb0, agg.at[pl.ds(zc * ZCH, ZCH)])

    plsc.subcore_barrier()

    sets = ((mb0, ls0), (mb1, ls1))

    def issue(j, mb, ls):
        pltpu.async_copy(msg.at[pl.ds(base + j * CH, CH)], mb, ls)

    def step(j, mb, ls):
        pltpu.make_async_copy(msg.at[pl.ds(base + j * CH, CH)], mb, ls).wait()
        pltpu.sync_copy(mb, agg.at[idxd.at[j]], add=True)

        @pl.when(j + 2 < NCHUNK)
        def _():
            issue(j + 2, mb, ls)

    issue(0, *sets[0])
    issue(1, *sets[1])

    def pair(j2, carry):
        step(2 * j2, *sets[0])
        step(2 * j2 + 1, *sets[1])
        return carry

    lax.fori_loop(0, NCHUNK // 2, pair, 0)
    step(NCHUNK - 1, *sets[0])

    plsc.subcore_barrier()

    for k in range(ZPT):
        wc = sid * ZPT + k

        @pl.when(wc < NZC)
        def _():
            pltpu.sync_copy(agg.at[pl.ds(wc * ZCH, ZCH)],
                            out.at[cid, pl.ds(wc * ZCH, ZCH)])


@functools.cache
def _scatter_kernel():
    return pl.kernel(
        _scatter_body,
        out_type=jax.ShapeDtypeStruct((NC, N, D), jnp.float32),
        mesh=_mesh(),
        scratch_types=[
            pltpu.VMEM((NCHUNK, CH), jnp.int32),
            pltpu.VMEM((CH, D), jnp.float32),
            pltpu.VMEM((CH, D), jnp.float32),
            pltpu.VMEM_SHARED((N, D), jnp.float32),
            pltpu.SemaphoreType.DMA,
            pltpu.SemaphoreType.DMA,
        ],
    )


def _scatter(m, dst3):
    return _scatter_kernel()(m, dst3)


def _pack_bf16(hi_f32, lo_f32):
    lo = lax.shift_right_logical(
        lax.bitcast_convert_type(
            lo_f32.astype(jnp.bfloat16).astype(jnp.float32), jnp.int32), 16)
    hi = lax.bitcast_convert_type(
        hi_f32.astype(jnp.bfloat16).astype(jnp.float32), jnp.int32)
    return jnp.bitwise_or(jnp.bitwise_and(hi, jnp.int32(-65536)), lo)


def _pre_body(h_ref, wd_ref, ws_ref, od_ref, os_ref):
    h = h_ref[...]
    pd = jnp.dot(h, wd_ref[...], preferred_element_type=jnp.float32)
    ps = jnp.dot(h, ws_ref[...], preferred_element_type=jnp.float32)
    od_ref[...] = _pack_bf16(pd[:, D:], pd[:, :D])
    os_ref[...] = _pack_bf16(ps[:, D:], ps[:, :D])


def _pre(h, wd, ws):
    blk = 2000
    return pl.pallas_call(
        _pre_body,
        grid=(N // blk,),
        in_specs=[
            pl.BlockSpec((blk, D), lambda i: (i, 0)),
            pl.BlockSpec((D, 2 * D), lambda i: (0, 0)),
            pl.BlockSpec((D, 2 * D), lambda i: (0, 0)),
        ],
        out_specs=[
            pl.BlockSpec((blk, D), lambda i: (i, 0)),
            pl.BlockSpec((blk, D), lambda i: (i, 0)),
        ],
        out_shape=[
            jax.ShapeDtypeStruct((N, D), jnp.int32),
            jax.ShapeDtypeStruct((N, D), jnp.int32),
        ],
    )(h, wd, ws)


def _msg_body(s_ref, ea_ref, we_ref, b_ref, o_ref):
    c = jnp.dot(ea_ref[...], we_ref[...], preferred_element_type=jnp.float32)
    c = c + b_ref[...]
    s32 = s_ref[...]
    zf = lax.bitcast_convert_type(lax.shift_left(s32, 16), jnp.float32)
    zs = lax.bitcast_convert_type(
        jnp.bitwise_and(s32, jnp.int32(-65536)), jnp.float32)
    zf = zf + c[:, :D]
    zs = zs + c[:, D:]
    sig = 1.0 / (1.0 + jnp.exp(-zf))
    sp = jnp.maximum(zs, 0.0) + jnp.log1p(jnp.exp(-jnp.abs(zs)))
    o_ref[...] = sig * sp


def _msg(s, ea, we, b):
    blk = 1000
    return pl.pallas_call(
        _msg_body,
        grid=(E // blk,),
        in_specs=[
            pl.BlockSpec((blk, D), lambda i: (i, 0)),
            pl.BlockSpec((blk, DE), lambda i: (i, 0)),
            pl.BlockSpec((DE, 2 * D), lambda i: (0, 0)),
            pl.BlockSpec((1, 2 * D), lambda i: (0, 0)),
        ],
        out_specs=pl.BlockSpec((blk, D), lambda i: (i, 0)),
        out_shape=jax.ShapeDtypeStruct((E, D), jnp.float32),
    )(s, ea, we, b)


def _bn_body(h_ref, agg_ref, g_ref, b_ref, o_ref):
    h = h_ref[...] + agg_ref[0] + agg_ref[1]
    mean = jnp.mean(h, axis=0, keepdims=True)
    var = jnp.mean((h - mean) ** 2, axis=0, keepdims=True)
    o_ref[...] = (h - mean) * lax.rsqrt(var + 1e-5) * g_ref[...] + b_ref[...]


def _bn(h, agg, g, b):
    return pl.pallas_call(
        _bn_body,
        in_specs=[
            pl.BlockSpec((N, D), lambda: (0, 0)),
            pl.BlockSpec((NC, N, D), lambda: (0, 0, 0)),
            pl.BlockSpec((1, D), lambda: (0, 0)),
            pl.BlockSpec((1, D), lambda: (0, 0)),
        ],
        out_specs=pl.BlockSpec((N, D), lambda: (0, 0)),
        out_shape=jax.ShapeDtypeStruct((N, D), jnp.float32),
    )(h, agg, g, b)


def _final_body(h_ref, agg_ref, g_ref, b_ref, bat_ref, o_ref):
    h = h_ref[...] + agg_ref[0] + agg_ref[1]
    mean = jnp.mean(h, axis=0, keepdims=True)
    var = jnp.mean((h - mean) ** 2, axis=0, keepdims=True)
    hn = (h - mean) * lax.rsqrt(var + 1e-5) * g_ref[...] + b_ref[...]
    onehot = (bat_ref[...] == lax.broadcasted_iota(jnp.int32, (1, B), 1))
    onehot = onehot.astype(jnp.float32)
    dn = (((0,), (0,)), ((), ()))
    sums = lax.dot_general(onehot, hn, dn, preferred_element_type=jnp.float32)
    cnt = lax.dot_general(onehot, jnp.ones((N, 1), jnp.float32), dn,
                          preferred_element_type=jnp.float32)
    o_ref[...] = sums / jnp.maximum(cnt, 1.0)


def _final(h, agg, g, b, bat):
    return pl.pallas_call(
        _final_body,
        in_specs=[
            pl.BlockSpec((N, D), lambda: (0, 0)),
            pl.BlockSpec((NC, N, D), lambda: (0, 0, 0)),
            pl.BlockSpec((1, D), lambda: (0, 0)),
            pl.BlockSpec((1, D), lambda: (0, 0)),
            pl.BlockSpec((N, 1), lambda: (0, 0)),
        ],
        out_specs=pl.BlockSpec((B, D), lambda: (0, 0)),
        out_shape=jax.ShapeDtypeStruct((B, D), jnp.float32),
    )(h, agg, g, b, bat)


def kernel(x, edge_index, edge_attr, batch,
           Wf0, bf0, Ws0, bs0, gamma0, beta0,
           Wf1, bf1, Ws1, bs1, gamma1, beta1):
    src3 = edge_index[0].reshape(NW, NCHUNK, CH)
    dst3 = edge_index[1].reshape(NW, NCHUNK, CH)

    def layer(h, Wf, bf, Ws, bs):
        wd = jnp.concatenate([Wf[:D], Ws[:D]], axis=1)
        wsr = jnp.concatenate([Wf[D:2 * D], Ws[D:2 * D]], axis=1)
        we = jnp.concatenate([Wf[2 * D:], Ws[2 * D:]], axis=1)
        be = jnp.concatenate([bf, bs])[None, :]
        td, ts = _pre(h, wd, wsr)
        s = _gather(td, ts, dst3, src3)
        m = _msg(s, edge_attr, we, be)
        return _scatter(m, dst3)

    a0 = layer(x, Wf0, bf0, Ws0, bs0)
    h1 = _bn(x, a0, gamma0[None, :], beta0[None, :])
    a1 = layer(h1, Wf1, bf1, Ws1, bs1)
    return _final(h1, a1, gamma1[None, :], beta1[None, :], batch[:, None])


# speedup vs baseline: 1.5303x; 1.5303x over previous
import functools

import jax
import jax.numpy as jnp
from jax import lax
from jax.experimental import pallas as pl
from jax.experimental.pallas import tpu as pltpu
from jax.experimental.pallas import tpu_sc as plsc

N = 10000
E = 320000
D = 128
DE = 16
B = 64

NC = 2
NS = 16
NW = NC * NS
EPW = E // NW
CH = 80
NCHUNK = EPW // CH
ZCH = 80
NZC = N // ZCH
ZPT = (NZC + NS - 1) // NS

@functools.cache
def _mesh():
    return plsc.VectorSubcoreMesh(core_axis_name="c", subcore_axis_name="s",
                                  num_cores=NC, num_subcores=NS)


def _gather_body(tdst, tsrc, dst3, src3, s_out,
                 idxd, idxs, ba0, bb0, ba1, bb1, ob0, ob1,
                 sa0, sb0, sa1, sb1, sw0, sw1):
    wid = lax.axis_index("s") * NC + lax.axis_index("c")
    base = wid * EPW

    pltpu.sync_copy(dst3.at[wid], idxd)
    pltpu.sync_copy(src3.at[wid], idxs)

    sets = ((ba0, bb0, ob0, sa0, sb0, sw0), (ba1, bb1, ob1, sa1, sb1, sw1))

    def issue(j, ba, bb, sa, sb):
        pltpu.async_copy(tdst.at[idxd.at[j]], ba, sa)
        pltpu.async_copy(tsrc.at[idxs.at[j]], bb, sb)

    def step(j, ba, bb, ob, sa, sb, sw):
        pltpu.make_async_copy(tdst.at[idxd.at[j]], ba, sa).wait()
        pltpu.make_async_copy(tsrc.at[idxs.at[j]], bb, sb).wait()

        @pl.when(j >= 2)
        def _():
            pltpu.make_async_copy(
                ob, s_out.at[pl.ds(base + (j - 2) * CH, CH)], sw).wait()

        mhi = jnp.int32(-65536)
        half = jnp.int32(32768)

        def row(r, c2):
            for c in range(D // 16):
                sl = pl.ds(c * 16, 16)
                va = ba[r, sl]
                vb = bb[r, sl]
                alo = lax.bitcast_convert_type(lax.shift_left(va, 16),
                                               jnp.float32)
                ahi = lax.bitcast_convert_type(jnp.bitwise_and(va, mhi),
                                               jnp.float32)
                blo = lax.bitcast_convert_type(lax.shift_left(vb, 16),
                                               jnp.float32)
                bhi = lax.bitcast_convert_type(jnp.bitwise_and(vb, mhi),
                                               jnp.float32)
                slo = lax.bitcast_convert_type(alo + blo, jnp.int32)
                shi = lax.bitcast_convert_type(ahi + bhi, jnp.int32)
                lo16 = lax.shift_right_logical(slo + half, 16)
                hi16 = jnp.bitwise_and(shi + half, mhi)
                ob[r, sl] = jnp.bitwise_or(hi16, lo16)
            return c2

        lax.fori_loop(0, CH, row, 0)

        @pl.when(j + 2 < NCHUNK)
        def _():
            issue(j + 2, ba, bb, sa, sb)

        pltpu.async_copy(ob, s_out.at[pl.ds(base + j * CH, CH)], sw)

    issue(0, *sets[0][:2], *sets[0][3:5])
    issue(1, *sets[1][:2], *sets[1][3:5])

    def pair(j2, carry):
        step(2 * j2, *sets[0])
        step(2 * j2 + 1, *sets[1])
        return carry

    lax.fori_loop(0, NCHUNK // 2, pair, 0)
    step(NCHUNK - 1, *sets[0])

    pltpu.make_async_copy(
        ob1, s_out.at[pl.ds(base + (NCHUNK - 2) * CH, CH)], sw1).wait()
    pltpu.make_async_copy(
        ob0, s_out.at[pl.ds(base + (NCHUNK - 1) * CH, CH)], sw0).wait()


@functools.cache
def _gather_kernel():
    return pl.kernel(
        _gather_body,
        out_type=jax.ShapeDtypeStruct((E, D), jnp.int32),
        mesh=_mesh(),
        scratch_types=[
            pltpu.VMEM((NCHUNK, CH), jnp.int32),
            pltpu.VMEM((NCHUNK, CH), jnp.int32),
            pltpu.VMEM((CH, D), jnp.int32),
            pltpu.VMEM((CH, D), jnp.int32),
            pltpu.VMEM((CH, D), jnp.int32),
            pltpu.VMEM((CH, D), jnp.int32),
            pltpu.VMEM((CH, D), jnp.int32),
            pltpu.VMEM((CH, D), jnp.int32),
            pltpu.SemaphoreType.DMA,
            pltpu.SemaphoreType.DMA,
            pltpu.SemaphoreType.DMA,
            pltpu.SemaphoreType.DMA,
            pltpu.SemaphoreType.DMA,
            pltpu.SemaphoreType.DMA,
        ],
    )


def _gather(td, ts, dst3, src3):
    return _gather_kernel()(td, ts, dst3, src3)


def _scatter_body(msg, dst3, out, idxd, mb0, mb1, agg, ls0, ls1):
    cid = lax.axis_index("c")
    sid = lax.axis_index("s")
    wid = sid * NC + cid
    base = wid * EPW

    pltpu.sync_copy(dst3.at[wid], idxd)

    def zrow(r, c2):
        for c in range(D // 16):
            mb0[r, pl.ds(c * 16, 16)] = jnp.zeros((16,), jnp.float32)
        return c2

    lax.fori_loop(0, ZCH, zrow, 0)
    for k in range(ZPT):
        zc = sid * ZPT + k

        @pl.when(zc < NZC)
        def _():
            pltpu.sync_copy(mb0, agg.at[pl.ds(zc * ZCH, ZCH)])

    plsc.subcore_barrier()

    sets = ((mb0, ls0), (mb1, ls1))

    def issue(j, mb, ls):
        pltpu.async_copy(msg.at[pl.ds(base + j * CH, CH)], mb, ls)

    def step(j, mb, ls):
        pltpu.make_async_copy(msg.at[pl.ds(base + j * CH, CH)], mb, ls).wait()
        pltpu.sync_copy(mb, agg.at[idxd.at[j]], add=True)

        @pl.when(j + 2 < NCHUNK)
        def _():
            issue(j + 2, mb, ls)

    issue(0, *sets[0])
    issue(1, *sets[1])

    def pair(j2, carry):
        step(2 * j2, *sets[0])
        step(2 * j2 + 1, *sets[1])
        return carry

    lax.fori_loop(0, NCHUNK // 2, pair, 0)
    step(NCHUNK - 1, *sets[0])

    plsc.subcore_barrier()

    for k in range(ZPT):
        wc = sid * ZPT + k

        @pl.when(wc < NZC)
        def _():
            pltpu.sync_copy(agg.at[pl.ds(wc * ZCH, ZCH)],
                            out.at[cid, pl.ds(wc * ZCH, ZCH)])


@functools.cache
def _scatter_kernel():
    return pl.kernel(
        _scatter_body,
        out_type=jax.ShapeDtypeStruct((NC, N, D), jnp.float32),
        mesh=_mesh(),
        scratch_types=[
            pltpu.VMEM((NCHUNK, CH), jnp.int32),
            pltpu.VMEM((CH, D), jnp.float32),
            pltpu.VMEM((CH, D), jnp.float32),
            pltpu.VMEM_SHARED((N, D), jnp.float32),
            pltpu.SemaphoreType.DMA,
            pltpu.SemaphoreType.DMA,
        ],
    )


def _scatter(m, dst3):
    return _scatter_kernel()(m, dst3)


def _pack_bf16(hi_f32, lo_f32):
    lo = lax.shift_right_logical(
        lax.bitcast_convert_type(
            lo_f32.astype(jnp.bfloat16).astype(jnp.float32), jnp.int32), 16)
    hi = lax.bitcast_convert_type(
        hi_f32.astype(jnp.bfloat16).astype(jnp.float32), jnp.int32)
    return jnp.bitwise_or(jnp.bitwise_and(hi, jnp.int32(-65536)), lo)


def _pre_body(h_ref, wd_ref, ws_ref, od_ref, os_ref):
    h = h_ref[...]
    pd = jnp.dot(h, wd_ref[...], preferred_element_type=jnp.float32)
    ps = jnp.dot(h, ws_ref[...], preferred_element_type=jnp.float32)
    od_ref[...] = _pack_bf16(pd[:, D:], pd[:, :D])
    os_ref[...] = _pack_bf16(ps[:, D:], ps[:, :D])


def _pre(h, wd, ws):
    blk = 2000
    return pl.pallas_call(
        _pre_body,
        grid=(N // blk,),
        in_specs=[
            pl.BlockSpec((blk, D), lambda i: (i, 0)),
            pl.BlockSpec((D, 2 * D), lambda i: (0, 0)),
            pl.BlockSpec((D, 2 * D), lambda i: (0, 0)),
        ],
        out_specs=[
            pl.BlockSpec((blk, D), lambda i: (i, 0)),
            pl.BlockSpec((blk, D), lambda i: (i, 0)),
        ],
        out_shape=[
            jax.ShapeDtypeStruct((N, D), jnp.int32),
            jax.ShapeDtypeStruct((N, D), jnp.int32),
        ],
    )(h, wd, ws)


def _msg_body(s_ref, ea_ref, we_ref, b_ref, o_ref):
    c = jnp.dot(ea_ref[...], we_ref[...], preferred_element_type=jnp.float32)
    c = c + b_ref[...]
    s32 = s_ref[...]
    zf = lax.bitcast_convert_type(lax.shift_left(s32, 16), jnp.float32)
    zs = lax.bitcast_convert_type(
        jnp.bitwise_and(s32, jnp.int32(-65536)), jnp.float32)
    zf = zf + c[:, :D]
    zs = zs + c[:, D:]
    sig = 1.0 / (1.0 + jnp.exp(-zf))
    sp = jnp.maximum(zs, 0.0) + jnp.log1p(jnp.exp(-jnp.abs(zs)))
    o_ref[...] = sig * sp


def _msg(s, ea, we, b):
    blk = 1000
    return pl.pallas_call(
        _msg_body,
        grid=(E // blk,),
        in_specs=[
            pl.BlockSpec((blk, D), lambda i: (i, 0)),
            pl.BlockSpec((blk, DE), lambda i: (i, 0)),
            pl.BlockSpec((DE, 2 * D), lambda i: (0, 0)),
            pl.BlockSpec((1, 2 * D), lambda i: (0, 0)),
        ],
        out_specs=pl.BlockSpec((blk, D), lambda i: (i, 0)),
        out_shape=jax.ShapeDtypeStruct((E, D), jnp.float32),
    )(s, ea, we, b)


def _bn_body(h_ref, agg_ref, g_ref, b_ref, o_ref):
    h = h_ref[...] + agg_ref[0] + agg_ref[1]
    mean = jnp.mean(h, axis=0, keepdims=True)
    var = jnp.mean((h - mean) ** 2, axis=0, keepdims=True)
    o_ref[...] = (h - mean) * lax.rsqrt(var + 1e-5) * g_ref[...] + b_ref[...]


def _bn(h, agg, g, b):
    return pl.pallas_call(
        _bn_body,
        in_specs=[
            pl.BlockSpec((N, D), lambda: (0, 0)),
            pl.BlockSpec((NC, N, D), lambda: (0, 0, 0)),
            pl.BlockSpec((1, D), lambda: (0, 0)),
            pl.BlockSpec((1, D), lambda: (0, 0)),
        ],
        out_specs=pl.BlockSpec((N, D), lambda: (0, 0)),
        out_shape=jax.ShapeDtypeStruct((N, D), jnp.float32),
    )(h, agg, g, b)


def _final_body(h_ref, agg_ref, g_ref, b_ref, bat_ref, o_ref):
    h = h_ref[...] + agg_ref[0] + agg_ref[1]
    mean = jnp.mean(h, axis=0, keepdims=True)
    var = jnp.mean((h - mean) ** 2, axis=0, keepdims=True)
    hn = (h - mean) * lax.rsqrt(var + 1e-5) * g_ref[...] + b_ref[...]
    onehot = (bat_ref[...] == lax.broadcasted_iota(jnp.int32, (1, B), 1))
    onehot = onehot.astype(jnp.float32)
    dn = (((0,), (0,)), ((), ()))
    sums = lax.dot_general(onehot, hn, dn, preferred_element_type=jnp.float32)
    cnt = lax.dot_general(onehot, jnp.ones((N, 1), jnp.float32), dn,
                          preferred_element_type=jnp.float32)
    o_ref[...] = sums / jnp.maximum(cnt, 1.0)


def _final(h, agg, g, b, bat):
    return pl.pallas_call(
        _final_body,
        in_specs=[
            pl.BlockSpec((N, D), lambda: (0, 0)),
            pl.BlockSpec((NC, N, D), lambda: (0, 0, 0)),
            pl.BlockSpec((1, D), lambda: (0, 0)),
            pl.BlockSpec((1, D), lambda: (0, 0)),
            pl.BlockSpec((N, 1), lambda: (0, 0)),
        ],
        out_specs=pl.BlockSpec((B, D), lambda: (0, 0)),
        out_shape=jax.ShapeDtypeStruct((B, D), jnp.float32),
    )(h, agg, g, b, bat)


def kernel(x, edge_index, edge_attr, batch,
           Wf0, bf0, Ws0, bs0, gamma0, beta0,
           Wf1, bf1, Ws1, bs1, gamma1, beta1):
    src3 = edge_index[0].reshape(NW, NCHUNK, CH)
    dst3 = edge_index[1].reshape(NW, NCHUNK, CH)

    def layer(h, Wf, bf, Ws, bs):
        wd = jnp.concatenate([Wf[:D], Ws[:D]], axis=1)
        wsr = jnp.concatenate([Wf[D:2 * D], Ws[D:2 * D]], axis=1)
        we = jnp.concatenate([Wf[2 * D:], Ws[2 * D:]], axis=1)
        be = jnp.concatenate([bf, bs])[None, :]
        td, ts = _pre(h, wd, wsr)
        s = _gather(td, ts, dst3, src3)
        m = _msg(s, edge_attr, we, be)
        return _scatter(m, dst3)

    a0 = layer(x, Wf0, bf0, Ws0, bs0)
    h1 = _bn(x, a0, gamma0[None, :], beta0[None, :])
    a1 = layer(h1, Wf1, bf1, Ws1, bs1)
    return _final(h1, a1, gamma1[None, :], beta1[None, :], batch[:, None])


# speedup vs baseline: 1.7211x; 1.1246x over previous
import functools

import jax
import jax.numpy as jnp
from jax import lax
from jax.experimental import pallas as pl
from jax.experimental.pallas import tpu as pltpu
from jax.experimental.pallas import tpu_sc as plsc

N = 10000
E = 320000
D = 128
DE = 16
B = 64

NC = 2
NS = 16
NW = NC * NS
EPW = E // NW
CH = 80
NCHUNK = EPW // CH
CA = 64
CB = NCHUNK - CA
ZCH = 80
NZC = N // ZCH
ZPT = (NZC + NS - 1) // NS


@functools.cache
def _mesh():
    return plsc.VectorSubcoreMesh(core_axis_name="c", subcore_axis_name="s",
                                  num_cores=NC, num_subcores=NS)


@functools.cache
def _gather_kernel(nchunk):
    epw = nchunk * CH

    def body(tdst, tsrc, dst3, src3, s_out,
             idxd, idxs, ba0, bb0, ba1, bb1, ob0, ob1,
             sa0, sb0, sa1, sb1, sw0, sw1):
        wid = lax.axis_index("s") * NC + lax.axis_index("c")
        base = wid * epw

        pltpu.sync_copy(dst3.at[wid], idxd)
        pltpu.sync_copy(src3.at[wid], idxs)

        sets = ((ba0, bb0, ob0, sa0, sb0, sw0),
                (ba1, bb1, ob1, sa1, sb1, sw1))

        def issue(j, ba, bb, sa, sb):
            pltpu.async_copy(tdst.at[idxd.at[j]], ba, sa)
            pltpu.async_copy(tsrc.at[idxs.at[j]], bb, sb)

        def step(j, ba, bb, ob, sa, sb, sw):
            pltpu.make_async_copy(tdst.at[idxd.at[j]], ba, sa).wait()
            pltpu.make_async_copy(tsrc.at[idxs.at[j]], bb, sb).wait()

            @pl.when(j >= 2)
            def _():
                pltpu.make_async_copy(
                    ob, s_out.at[pl.ds(base + (j - 2) * CH, CH)], sw).wait()

            mhi = jnp.int32(-65536)
            half = jnp.int32(32768)

            def row(r, c2):
                for c in range(D // 16):
                    sl = pl.ds(c * 16, 16)
                    va = ba[r, sl]
                    vb = bb[r, sl]
                    alo = lax.bitcast_convert_type(lax.shift_left(va, 16),
                                                   jnp.float32)
                    ahi = lax.bitcast_convert_type(jnp.bitwise_and(va, mhi),
                                                   jnp.float32)
                    blo = lax.bitcast_convert_type(lax.shift_left(vb, 16),
                                                   jnp.float32)
                    bhi = lax.bitcast_convert_type(jnp.bitwise_and(vb, mhi),
                                                   jnp.float32)
                    slo = lax.bitcast_convert_type(alo + blo, jnp.int32)
                    shi = lax.bitcast_convert_type(ahi + bhi, jnp.int32)
                    lo16 = lax.shift_right_logical(slo + half, 16)
                    hi16 = jnp.bitwise_and(shi + half, mhi)
                    ob[r, sl] = jnp.bitwise_or(hi16, lo16)
                return c2

            lax.fori_loop(0, CH, row, 0)

            @pl.when(j + 2 < nchunk)
            def _():
                issue(j + 2, ba, bb, sa, sb)

            pltpu.async_copy(ob, s_out.at[pl.ds(base + j * CH, CH)], sw)

        issue(0, *sets[0][:2], *sets[0][3:5])
        issue(1, *sets[1][:2], *sets[1][3:5])

        def pair(j2, carry):
            step(2 * j2, *sets[0])
            step(2 * j2 + 1, *sets[1])
            return carry

        lax.fori_loop(0, nchunk // 2, pair, 0)
        if nchunk % 2:
            step(nchunk - 1, *sets[0])

        s_pen = sets[(nchunk - 2) % 2]
        s_fin = sets[(nchunk - 1) % 2]
        pltpu.make_async_copy(
            s_pen[2], s_out.at[pl.ds(base + (nchunk - 2) * CH, CH)],
            s_pen[5]).wait()
        pltpu.make_async_copy(
            s_fin[2], s_out.at[pl.ds(base + (nchunk - 1) * CH, CH)],
            s_fin[5]).wait()

    return pl.kernel(
        body,
        out_type=jax.ShapeDtypeStruct((NW * epw, D), jnp.int32),
        mesh=_mesh(),
        scratch_types=[
            pltpu.VMEM((nchunk, CH), jnp.int32),
            pltpu.VMEM((nchunk, CH), jnp.int32),
            pltpu.VMEM((CH, D), jnp.int32),
            pltpu.VMEM((CH, D), jnp.int32),
            pltpu.VMEM((CH, D), jnp.int32),
            pltpu.VMEM((CH, D), jnp.int32),
            pltpu.VMEM((CH, D), jnp.int32),
            pltpu.VMEM((CH, D), jnp.int32),
            pltpu.SemaphoreType.DMA,
            pltpu.SemaphoreType.DMA,
            pltpu.SemaphoreType.DMA,
            pltpu.SemaphoreType.DMA,
            pltpu.SemaphoreType.DMA,
            pltpu.SemaphoreType.DMA,
        ],
    )


def _gather(td, ts, dst3, src3, nchunk):
    return _gather_kernel(nchunk)(td, ts, dst3, src3)


@functools.cache
def _scatter_kernel(nchunk):
    epw = nchunk * CH

    def body(msg, dst3, out, idxd, mb0, mb1, agg, ls0, ls1):
        cid = lax.axis_index("c")
        sid = lax.axis_index("s")
        wid = sid * NC + cid
        base = wid * epw

        pltpu.sync_copy(dst3.at[wid], idxd)

        def zrow(r, c2):
            for c in range(D // 16):
                mb0[r, pl.ds(c * 16, 16)] = jnp.zeros((16,), jnp.float32)
            return c2

        lax.fori_loop(0, ZCH, zrow, 0)
        for k in range(ZPT):
            zc = sid * ZPT + k

            @pl.when(zc < NZC)
            def _():
                pltpu.sync_copy(mb0, agg.at[pl.ds(zc * ZCH, ZCH)])

        plsc.subcore_barrier()

        sets = ((mb0, ls0), (mb1, ls1))

        def issue(j, mb, ls):
            pltpu.async_copy(msg.at[pl.ds(base + j * CH, CH)], mb, ls)

        def step(j, mb, ls):
            pltpu.make_async_copy(
                msg.at[pl.ds(base + j * CH, CH)], mb, ls).wait()
            pltpu.sync_copy(mb, agg.at[idxd.at[j]], add=True)

            @pl.when(j + 2 < nchunk)
            def _():
                issue(j + 2, mb, ls)

        issue(0, *sets[0])
        issue(1, *sets[1])

        def pair(j2, carry):
            step(2 * j2, *sets[0])
            step(2 * j2 + 1, *sets[1])
            return carry

        lax.fori_loop(0, nchunk // 2, pair, 0)
        if nchunk % 2:
            step(nchunk - 1, *sets[0])

        plsc.subcore_barrier()

        for k in range(ZPT):
            wc = sid * ZPT + k

            @pl.when(wc < NZC)
            def _():
                pltpu.sync_copy(agg.at[pl.ds(wc * ZCH, ZCH)],
                                out.at[cid, pl.ds(wc * ZCH, ZCH)])

    return pl.kernel(
        body,
        out_type=jax.ShapeDtypeStruct((NC, N, D), jnp.float32),
        mesh=_mesh(),
        scratch_types=[
            pltpu.VMEM((nchunk, CH), jnp.int32),
            pltpu.VMEM((CH, D), jnp.float32),
            pltpu.VMEM((CH, D), jnp.float32),
            pltpu.VMEM_SHARED((N, D), jnp.float32),
            pltpu.SemaphoreType.DMA,
            pltpu.SemaphoreType.DMA,
        ],
    )


def _scatter(m, dst3, nchunk):
    return _scatter_kernel(nchunk)(m, dst3)


def _pack_bf16(hi_f32, lo_f32):
    lo = lax.shift_right_logical(
        lax.bitcast_convert_type(
            lo_f32.astype(jnp.bfloat16).astype(jnp.float32), jnp.int32), 16)
    hi = lax.bitcast_convert_type(
        hi_f32.astype(jnp.bfloat16).astype(jnp.float32), jnp.int32)
    return jnp.bitwise_or(jnp.bitwise_and(hi, jnp.int32(-65536)), lo)


def _pre_body(h_ref, wd_ref, ws_ref, od_ref, os_ref):
    h = h_ref[...]
    pd = jnp.dot(h, wd_ref[...], preferred_element_type=jnp.float32)
    ps = jnp.dot(h, ws_ref[...], preferred_element_type=jnp.float32)
    od_ref[...] = _pack_bf16(pd[:, D:], pd[:, :D])
    os_ref[...] = _pack_bf16(ps[:, D:], ps[:, :D])


def _pre(h, wd, ws):
    blk = 2000
    return pl.pallas_call(
        _pre_body,
        grid=(N // blk,),
        in_specs=[
            pl.BlockSpec((blk, D), lambda i: (i, 0)),
            pl.BlockSpec((D, 2 * D), lambda i: (0, 0)),
            pl.BlockSpec((D, 2 * D), lambda i: (0, 0)),
        ],
        out_specs=[
            pl.BlockSpec((blk, D), lambda i: (i, 0)),
            pl.BlockSpec((blk, D), lambda i: (i, 0)),
        ],
        out_shape=[
            jax.ShapeDtypeStruct((N, D), jnp.int32),
            jax.ShapeDtypeStruct((N, D), jnp.int32),
        ],
    )(h, wd, ws)


def _msg_body(s_ref, ea_ref, we_ref, b_ref, o_ref):
    c = jnp.dot(ea_ref[...], we_ref[...], preferred_element_type=jnp.float32)
    c = c + b_ref[...]
    s32 = s_ref[...]
    zf = lax.bitcast_convert_type(lax.shift_left(s32, 16), jnp.float32)
    zs = lax.bitcast_convert_type(
        jnp.bitwise_and(s32, jnp.int32(-65536)), jnp.float32)
    zf = zf + c[:, :D]
    zs = zs + c[:, D:]
    sig = 1.0 / (1.0 + jnp.exp(-zf))
    sp = jnp.maximum(zs, 0.0) + jnp.log1p(jnp.exp(-jnp.abs(zs)))
    o_ref[...] = sig * sp


def _msg(s, ea, we, b, blk):
    rows = s.shape[0]
    return pl.pallas_call(
        _msg_body,
        grid=(rows // blk,),
        in_specs=[
            pl.BlockSpec((blk, D), lambda i: (i, 0)),
            pl.BlockSpec((blk, DE), lambda i: (i, 0)),
            pl.BlockSpec((DE, 2 * D), lambda i: (0, 0)),
            pl.BlockSpec((1, 2 * D), lambda i: (0, 0)),
        ],
        out_specs=pl.BlockSpec((blk, D), lambda i: (i, 0)),
        out_shape=jax.ShapeDtypeStruct((rows, D), jnp.float32),
    )(s, ea, we, b)


def _bn_body(h_ref, aa_ref, ab_ref, g_ref, b_ref, o_ref):
    h = (h_ref[...] + aa_ref[0] + aa_ref[1]) + (ab_ref[0] + ab_ref[1])
    mean = jnp.mean(h, axis=0, keepdims=True)
    var = jnp.mean((h - mean) ** 2, axis=0, keepdims=True)
    o_ref[...] = (h - mean) * lax.rsqrt(var + 1e-5) * g_ref[...] + b_ref[...]


def _bn(h, aa, ab, g, b):
    return pl.pallas_call(
        _bn_body,
        in_specs=[
            pl.BlockSpec((N, D), lambda: (0, 0)),
            pl.BlockSpec((NC, N, D), lambda: (0, 0, 0)),
            pl.BlockSpec((NC, N, D), lambda: (0, 0, 0)),
            pl.BlockSpec((1, D), lambda: (0, 0)),
            pl.BlockSpec((1, D), lambda: (0, 0)),
        ],
        out_specs=pl.BlockSpec((N, D), lambda: (0, 0)),
        out_shape=jax.ShapeDtypeStruct((N, D), jnp.float32),
    )(h, aa, ab, g, b)


def _final_body(h_ref, aa_ref, ab_ref, g_ref, b_ref, bat_ref, o_ref):
    h = (h_ref[...] + aa_ref[0] + aa_ref[1]) + (ab_ref[0] + ab_ref[1])
    mean = jnp.mean(h, axis=0, keepdims=True)
    var = jnp.mean((h - mean) ** 2, axis=0, keepdims=True)
    hn = (h - mean) * lax.rsqrt(var + 1e-5) * g_ref[...] + b_ref[...]
    onehot = (bat_ref[...] == lax.broadcasted_iota(jnp.int32, (1, B), 1))
    onehot = onehot.astype(jnp.float32)
    dn = (((0,), (0,)), ((), ()))
    sums = lax.dot_general(onehot, hn, dn, preferred_element_type=jnp.float32)
    cnt = lax.dot_general(onehot, jnp.ones((N, 1), jnp.float32), dn,
                          preferred_element_type=jnp.float32)
    o_ref[...] = sums / jnp.maximum(cnt, 1.0)


def _final(h, aa, ab, g, b, bat):
    return pl.pallas_call(
        _final_body,
        in_specs=[
            pl.BlockSpec((N, D), lambda: (0, 0)),
            pl.BlockSpec((NC, N, D), lambda: (0, 0, 0)),
            pl.BlockSpec((NC, N, D), lambda: (0, 0, 0)),
            pl.BlockSpec((1, D), lambda: (0, 0)),
            pl.BlockSpec((1, D), lambda: (0, 0)),
            pl.BlockSpec((N, 1), lambda: (0, 0)),
        ],
        out_specs=pl.BlockSpec((B, D), lambda: (0, 0)),
        out_shape=jax.ShapeDtypeStruct((B, D), jnp.float32),
    )(h, aa, ab, g, b, bat)


def kernel(x, edge_index, edge_attr, batch,
           Wf0, bf0, Ws0, bs0, gamma0, beta0,
           Wf1, bf1, Ws1, bs1, gamma1, beta1):
    srcR = edge_index[0].reshape(NW, NCHUNK, CH)
    dstR = edge_index[1].reshape(NW, NCHUNK, CH)
    srcA, srcB = srcR[:, :CA], srcR[:, CA:]
    dstA, dstB = dstR[:, :CA], dstR[:, CA:]
    eaR = edge_attr.reshape(NW, NCHUNK, CH, DE)
    eaA = eaR[:, :CA].reshape(NW * CA * CH, DE)
    eaB = eaR[:, CA:].reshape(NW * CB * CH, DE)

    def layer(h, Wf, bf, Ws, bs):
        wd = jnp.concatenate([Wf[:D], Ws[:D]], axis=1)
        wsr = jnp.concatenate([Wf[D:2 * D], Ws[D:2 * D]], axis=1)
        we = jnp.concatenate([Wf[2 * D:], Ws[2 * D:]], axis=1)
        be = jnp.concatenate([bf, bs])[None, :]
        td, ts = _pre(h, wd, wsr)
        s_a = _gather(td, ts, dstA, srcA, CA)
        s_b = _gather(td, ts, dstB, srcB, CB)
        m_a = _msg(s_a, eaA, we, be, 1024)
        m_b = _msg(s_b, eaB, we, be, 976)
        agg_a = _scatter(m_a, dstA, CA)
        agg_b = _scatter(m_b, dstB, CB)
        return agg_a, agg_b

    aa0, ab0 = layer(x, Wf0, bf0, Ws0, bs0)
    h1 = _bn(x, aa0, ab0, gamma0[None, :], beta0[None, :])
    aa1, ab1 = layer(h1, Wf1, bf1, Ws1, bs1)
    return _final(h1, aa1, ab1, gamma1[None, :], beta1[None, :], batch[:, None])
